# baseline (speedup 1.0000x reference)
import jax
import jax.numpy as jnp
from jax.experimental import pallas as pl

_K = 8
_P = 2 * _K + 2
_MINW = 1e-3
_MINH = 1e-3
_LANES = 2048


def _edges(us, minv):
    e = [jnp.exp(u) for u in us]
    tot = e[0]
    for t in e[1:]:
        tot = tot + t
    scale = (1.0 - minv * _K) / tot
    w = [minv + t * scale for t in e]
    cum = [jnp.zeros_like(tot)]
    run = w[0]
    for k in range(_K - 1):
        cum.append(run)
        run = run + w[k + 1]
    cum.append(jnp.ones_like(tot))
    sizes = w[:-1] + [cum[_K] - cum[_K - 1]]
    return cum, sizes


def _spline_kernel(xl_ref, xu_ref, w_ref, b_ref, zl_ref, zu_ref):
    C = xl_ref.shape[0]
    n = xl_ref.shape[1] * xl_ref.shape[2]
    xl = xl_ref[...].reshape(C, n)
    xu = xu_ref[...].reshape(C, n)
    p = jnp.dot(w_ref[...], xl, preferred_element_type=jnp.float32) + b_ref[...]

    def slab(j):
        return p[j * C:(j + 1) * C, :]

    uw = [slab(k) for k in range(_K)]
    uh = [slab(_K + k) for k in range(_K)]
    udl = slab(2 * _K)
    udr = slab(2 * _K + 1)

    cw, wid = _edges(uw, _MINW)
    chh, hei = _edges(uh, _MINH)
    rw = [1.0 / wid[k] for k in range(_K)]
    s = [hei[k] * rw[k] for k in range(_K)]

    dv = [jax.nn.sigmoid(udl) * 3.0 * s[0]]
    for k in range(1, _K):
        sl, sr = s[k - 1], s[k]
        wl, wr = wid[k - 1], wid[k]
        m1 = jnp.minimum(sl, sr)
        m2 = 0.5 * (wr * sl + wl * sr) / (wl + wr)
        dv.append(2.0 * jnp.minimum(m1, m2))
    dv.append(jax.nn.sigmoid(udr) * 3.0 * s[_K - 1])

    def _eval(x):
        ss, dl, dr, rwx, dd, lo = s[0], dv[0], dv[1], rw[0], chh[0], cw[0]
        for k in range(1, _K):
            msk = x >= cw[k]
            ss = jnp.where(msk, s[k], ss)
            dl = jnp.where(msk, dv[k], dl)
            dr = jnp.where(msk, dv[k + 1], dr)
            rwx = jnp.where(msk, rw[k], rwx)
            dd = jnp.where(msk, chh[k], dd)
            lo = jnp.where(msk, cw[k], lo)
        sx = x - lo
        t1 = dl + dr
        rw2 = rwx * rwx
        ca = (t1 - 2.0 * ss) * rw2
        cb = (3.0 * ss - dl - t1) * rwx
        sx2 = sx * sx
        out = ca * (sx2 * sx) + cb * sx2 + dl * sx + dd
        return jnp.clip(out, 0.0, 1.0)

    shp = xl_ref.shape
    zl_ref[...] = _eval(xl).reshape(shp)
    zu_ref[...] = _eval(xu).reshape(shp)


_ROWS = _LANES // 128


def _run(xl3, xu3, Wp, bp, interpret=False):
    C, H, Wd = xl3.shape
    grid = H // _ROWS
    bx = pl.BlockSpec((C, _ROWS, Wd), lambda i: (0, i, 0))
    bw = pl.BlockSpec((_P * C, C), lambda i: (0, 0))
    bb = pl.BlockSpec((_P * C, 1), lambda i: (0, 0))
    return pl.pallas_call(
        _spline_kernel,
        grid=(grid,),
        in_specs=[bx, bx, bw, bb],
        out_specs=[bx, bx],
        out_shape=[jax.ShapeDtypeStruct((C, H, Wd), jnp.float32)] * 2,
        interpret=interpret,
    )(xl3, xu3, Wp, bp)


@jax.jit
def kernel(x_lower, x_upper, W, bconv):
    B, C, H, Wd = x_lower.shape
    xl3 = x_lower.reshape(C, H, Wd)
    xu3 = x_upper.reshape(C, H, Wd)
    Wp = W.reshape(C, _P, C).transpose(1, 0, 2).reshape(_P * C, C)
    bp = bconv.reshape(C, _P).T.reshape(_P * C, 1)
    zl3, zu3 = _run(xl3, xu3, Wp, bp)
    return zl3.reshape(B, C, H, Wd), zu3.reshape(B, C, H, Wd)


# speedup vs baseline: 1.0854x; 1.0854x over previous
import jax
import jax.numpy as jnp
from jax.experimental import pallas as pl

_K = 8
_P = 2 * _K + 2
_MINW = 1e-3
_MINH = 1e-3
_LANES = 1024


def _edges(us, minv):
    e = [jnp.exp(u) for u in us]
    tot = e[0]
    for t in e[1:]:
        tot = tot + t
    scale = (1.0 - minv * _K) / tot
    w = [minv + t * scale for t in e]
    cum = [jnp.zeros_like(tot)]
    run = w[0]
    for k in range(_K - 1):
        cum.append(run)
        run = run + w[k + 1]
    cum.append(jnp.ones_like(tot))
    sizes = w[:-1] + [cum[_K] - cum[_K - 1]]
    return cum, sizes


def _spline_kernel(xl_ref, xu_ref, w_ref, zl_ref, zu_ref):
    C = xl_ref.shape[0]
    n = xl_ref.shape[1] * xl_ref.shape[2]
    xl = xl_ref[...].reshape(C, n)
    xu = xu_ref[...].reshape(C, n)
    xa = jnp.concatenate([xl, jnp.ones((1, n), jnp.float32)], axis=0)
    p = jnp.dot(w_ref[...], xa, preferred_element_type=jnp.float32)

    def slab(j):
        return p[j * C:(j + 1) * C, :]

    uw = [slab(k) for k in range(_K)]
    uh = [slab(_K + k) for k in range(_K)]
    udl = slab(2 * _K)
    udr = slab(2 * _K + 1)

    cw, wid = _edges(uw, _MINW)
    chh, hei = _edges(uh, _MINH)
    rw = [1.0 / wid[k] for k in range(_K)]
    s = [hei[k] * rw[k] for k in range(_K)]

    dv = [jax.nn.sigmoid(udl) * 3.0 * s[0]]
    for k in range(1, _K):
        sl, sr = s[k - 1], s[k]
        wl, wr = wid[k - 1], wid[k]
        m1 = jnp.minimum(sl, sr)
        m2 = (wr * sl + wl * sr) / (wl + wr)
        dv.append(jnp.minimum(2.0 * m1, m2))
    dv.append(jax.nn.sigmoid(udr) * 3.0 * s[_K - 1])

    def _eval(x):
        ss, dl, dr, rwx, dd, lo = s[0], dv[0], dv[1], rw[0], chh[0], cw[0]
        for k in range(1, _K):
            msk = x >= cw[k]
            ss = jnp.where(msk, s[k], ss)
            dl = jnp.where(msk, dv[k], dl)
            dr = jnp.where(msk, dv[k + 1], dr)
            rwx = jnp.where(msk, rw[k], rwx)
            dd = jnp.where(msk, chh[k], dd)
            lo = jnp.where(msk, cw[k], lo)
        sx = x - lo
        t1 = dl + dr
        rw2 = rwx * rwx
        ca = (t1 - 2.0 * ss) * rw2
        cb = (3.0 * ss - dl - t1) * rwx
        sx2 = sx * sx
        out = ca * (sx2 * sx) + cb * sx2 + dl * sx + dd
        return jnp.clip(out, 0.0, 1.0)

    shp = xl_ref.shape
    zl_ref[...] = _eval(xl).reshape(shp)
    zu_ref[...] = _eval(xu).reshape(shp)


_ROWS = _LANES // 128


def _run(xl3, xu3, Wp, interpret=False):
    C, H, Wd = xl3.shape
    grid = H // _ROWS
    bx = pl.BlockSpec((C, _ROWS, Wd), lambda i: (0, i, 0))
    bw = pl.BlockSpec((_P * C, C + 1), lambda i: (0, 0))
    return pl.pallas_call(
        _spline_kernel,
        grid=(grid,),
        in_specs=[bx, bx, bw],
        out_specs=[bx, bx],
        out_shape=[jax.ShapeDtypeStruct((C, H, Wd), jnp.float32)] * 2,
        interpret=interpret,
    )(xl3, xu3, Wp)


@jax.jit
def kernel(x_lower, x_upper, W, bconv):
    B, C, H, Wd = x_lower.shape
    xl3 = x_lower.reshape(C, H, Wd)
    xu3 = x_upper.reshape(C, H, Wd)
    Wp = W.reshape(C, _P, C).transpose(1, 0, 2).reshape(_P * C, C)
    bp = bconv.reshape(C, _P).T.reshape(_P * C, 1)
    Wp = jnp.concatenate([Wp, bp], axis=1)
    zl3, zu3 = _run(xl3, xu3, Wp)
    return zl3.reshape(B, C, H, Wd), zu3.reshape(B, C, H, Wd)


# speedup vs baseline: 1.0868x; 1.0012x over previous
import jax
import jax.numpy as jnp
from jax.experimental import pallas as pl
from jax.experimental.pallas import tpu as pltpu

_K = 8
_P = 2 * _K + 2
_MINW = 1e-3
_MINH = 1e-3
_LANES = 1024


def _edges(e, tot, minv):
    scale = (1.0 - minv * _K) / tot
    w = [minv + t * scale for t in e]
    cum = [jnp.zeros_like(tot)]
    run = w[0]
    for k in range(_K - 1):
        cum.append(run)
        run = run + w[k + 1]
    cum.append(jnp.ones_like(tot))
    sizes = w[:-1] + [cum[_K] - cum[_K - 1]]
    return cum, sizes


def _spline_kernel(xl_ref, xu_ref, w_ref, zl_ref, zu_ref):
    C = xl_ref.shape[0]
    n = xl_ref.shape[1] * xl_ref.shape[2]
    xl = xl_ref[...].reshape(C, n)
    xu = xu_ref[...].reshape(C, n)
    xa = jnp.concatenate([xl, jnp.ones((1, n), jnp.float32)], axis=0)
    p = jnp.dot(w_ref[...], xa, preferred_element_type=jnp.float32)

    def slab(j):
        return p[j * C:(j + 1) * C, :]

    e_all = jnp.exp(p[0:2 * _K * C, :])
    ew = [e_all[k * C:(k + 1) * C, :] for k in range(_K)]
    eh = [e_all[(_K + k) * C:(_K + k + 1) * C, :] for k in range(_K)]
    udl = slab(2 * _K)
    udr = slab(2 * _K + 1)

    def _tot(e):
        tot = e[0]
        for t in e[1:]:
            tot = tot + t
        return tot

    cw, wid = _edges(ew, _tot(ew), _MINW)
    chh, hei = _edges(eh, _tot(eh), _MINH)
    rw = [1.0 / wid[k] for k in range(_K)]
    s = [hei[k] * rw[k] for k in range(_K)]

    dv = [jax.nn.sigmoid(udl) * 3.0 * s[0]]
    for k in range(1, _K):
        sl, sr = s[k - 1], s[k]
        wl, wr = wid[k - 1], wid[k]
        m1 = jnp.minimum(sl, sr)
        m2 = (wr * sl + wl * sr) / (wl + wr)
        dv.append(jnp.minimum(2.0 * m1, m2))
    dv.append(jax.nn.sigmoid(udr) * 3.0 * s[_K - 1])

    def _eval(x):
        ss, dl, dr, rwx, dd, lo = s[0], dv[0], dv[1], rw[0], chh[0], cw[0]
        for k in range(1, _K):
            msk = x >= cw[k]
            ss = jnp.where(msk, s[k], ss)
            dl = jnp.where(msk, dv[k], dl)
            dr = jnp.where(msk, dv[k + 1], dr)
            rwx = jnp.where(msk, rw[k], rwx)
            dd = jnp.where(msk, chh[k], dd)
            lo = jnp.where(msk, cw[k], lo)
        sx = x - lo
        t1 = dl + dr
        rw2 = rwx * rwx
        ca = (t1 - 2.0 * ss) * rw2
        cb = (3.0 * ss - dl - t1) * rwx
        out = ((ca * sx + cb) * sx + dl) * sx + dd
        return jnp.clip(out, 0.0, 1.0)

    shp = xl_ref.shape
    zl_ref[...] = _eval(xl).reshape(shp)
    zu_ref[...] = _eval(xu).reshape(shp)


_ROWS = _LANES // 128


def _run(xl3, xu3, Wp, interpret=False):
    C, H, Wd = xl3.shape
    grid = H // _ROWS
    bx = pl.BlockSpec((C, _ROWS, Wd), lambda i: (0, i, 0))
    bw = pl.BlockSpec((_P * C, C + 1), lambda i: (0, 0))
    return pl.pallas_call(
        _spline_kernel,
        grid=(grid,),
        in_specs=[bx, bx, bw],
        out_specs=[bx, bx],
        out_shape=[jax.ShapeDtypeStruct((C, H, Wd), jnp.float32)] * 2,
        compiler_params=None if interpret else pltpu.CompilerParams(
            dimension_semantics=(pltpu.PARALLEL,)),
        interpret=interpret,
    )(xl3, xu3, Wp)


@jax.jit
def kernel(x_lower, x_upper, W, bconv):
    B, C, H, Wd = x_lower.shape
    xl3 = x_lower.reshape(C, H, Wd)
    xu3 = x_upper.reshape(C, H, Wd)
    Wp = W.reshape(C, _P, C).transpose(1, 0, 2).reshape(_P * C, C)
    bp = bconv.reshape(C, _P).T.reshape(_P * C, 1)
    Wp = jnp.concatenate([Wp, bp], axis=1)
    zl3, zu3 = _run(xl3, xu3, Wp)
    return zl3.reshape(B, C, H, Wd), zu3.reshape(B, C, H, Wd)
